# baseline (speedup 1.0000x reference)
import functools

import jax
import jax.numpy as jnp
from jax import lax
from jax.experimental import pallas as pl
from jax.experimental.pallas import tpu as pltpu
from jax.experimental.pallas import tpu_sc as plsc

B = 16384
C = 1001
NADV = 1000
S = 1.0 / NADV

NC, NS, L = 2, 16, 16
NW = NC * NS
PW = B // NW
CH = 32
NCH = PW // CH
NG = CH // L


def _make_sc():
    mesh = plsc.VectorSubcoreMesh(core_axis_name="c", subcore_axis_name="s")

    @functools.partial(
        pl.kernel,
        out_type=(jax.ShapeDtypeStruct((B // 2,), jnp.float32),
                  jax.ShapeDtypeStruct((B // 2,), jnp.float32)),
        mesh=mesh,
        compiler_params=pltpu.CompilerParams(needs_layout_passes=False),
        scratch_types=[
            pltpu.VMEM((CH, C), jnp.float32),
            pltpu.VMEM((CH, C), jnp.float32),
            pltpu.VMEM((PW,), jnp.int32),
            pltpu.VMEM((PW,), jnp.float32),
            pltpu.SemaphoreType.DMA((2,)),
            pltpu.SemaphoreType.DMA,
        ],
    )
    def sc_duelqa(x_hbm, a_hbm, out0_hbm, out1_hbm, xv0, xv1, av, ov, sems, asem):
        core = lax.axis_index("c")
        sid = lax.axis_index("s")
        lbase = sid * PW
        base = core * (B // 2) + lbase
        pltpu.async_copy(a_hbm.at[pl.ds(base, PW)], av, asem).wait()
        lane = lax.iota(jnp.int32, L)
        zero16 = jnp.zeros((L,), jnp.float32)
        tailm = (lane >= 8).astype(jnp.float32)

        def cp(c, b):
            return pltpu.make_async_copy(
                x_hbm.at[pl.ds(base + c * CH, CH), :],
                xv0 if b == 0 else xv1,
                sems.at[b],
            )

        cp(0, 0).start()
        cp(1, 1).start()

        def _chunk(c, b):
            cp(c, b).wait()
            xb = xv0 if b == 0 else xv1
            for g in range(NG):
                lrows = g * L + lane

                def row_body(r, tvec):
                    row = g * L + r
                    accs = [zero16, zero16, zero16, zero16]
                    for j in range(62):
                        accs[j % 4] = accs[j % 4] + xb[row, pl.ds(j * L, L)]
                    tail = xb[row, pl.ds(984, L)] * tailm
                    acc = (accs[0] + accs[1]) + (accs[2] + accs[3]) + tail
                    t = jnp.sum(acc)
                    oh = (lane == r).astype(jnp.float32)
                    return tvec + t * oh

                tvec = lax.fori_loop(0, L, row_body, zero16)
                off = c * CH + g * L
                a16 = av[pl.ds(off, L)]
                gv = plsc.load_gather(xb, [lrows, a16])
                ov[pl.ds(off, L)] = gv - tvec * jnp.float32(S)

            @pl.when(c + 2 < NCH)
            def _():
                cp(c + 2, b).start()

        def pair_body(pair, carry):
            for b in range(2):
                _chunk(pair * 2 + b, b)
            return carry

        lax.fori_loop(0, NCH // 2, pair_body, 0)

        @pl.when(core == 0)
        def _():
            pltpu.sync_copy(ov, out0_hbm.at[pl.ds(lbase, PW)])

        @pl.when(core == 1)
        def _():
            pltpu.sync_copy(ov, out1_hbm.at[pl.ds(lbase, PW)])

    return sc_duelqa


_SC = _make_sc()


def kernel(x, a):
    a32 = a.reshape(-1).astype(jnp.int32)
    p0, p1 = _SC(x, a32)
    partial = jnp.concatenate([p0, p1])
    return (partial + x[:, NADV])[:, None]


# speedup vs baseline: 1.0064x; 1.0064x over previous
import functools

import jax
import jax.numpy as jnp
from jax import lax
from jax.experimental import pallas as pl
from jax.experimental.pallas import tpu as pltpu
from jax.experimental.pallas import tpu_sc as plsc

B = 16384
C = 1001
NADV = 1000
S = 1.0 / NADV

NC, NS, L = 2, 16, 16
NW = NC * NS
PW = B // NW
CH = 32
NCH = PW // CH
NG = CH // L


def _make_sc():
    mesh = plsc.VectorSubcoreMesh(core_axis_name="c", subcore_axis_name="s")

    @functools.partial(
        pl.kernel,
        out_type=jax.ShapeDtypeStruct((B,), jnp.float32),
        mesh=mesh,
        compiler_params=pltpu.CompilerParams(needs_layout_passes=False),
        scratch_types=[
            pltpu.VMEM((CH, C), jnp.float32),
            pltpu.VMEM((CH, C), jnp.float32),
            pltpu.VMEM((CH, C), jnp.float32),
            pltpu.VMEM((PW,), jnp.int32),
            pltpu.VMEM((PW,), jnp.float32),
            pltpu.SemaphoreType.DMA((3,)),
            pltpu.SemaphoreType.DMA,
        ],
    )
    def sc_duelqa(x_hbm, a_hbm, out_hbm, xv0, xv1, xv2, av, ov, sems, asem):
        wid = lax.axis_index("s") * NC + lax.axis_index("c")
        base = wid * PW
        pltpu.async_copy(a_hbm.at[pl.ds(base, PW)], av, asem).wait()
        lane = lax.iota(jnp.int32, L)
        zero16 = jnp.zeros((L,), jnp.float32)
        tailm = (lane >= 8).astype(jnp.float32)

        bufs = [xv0, xv1, xv2]

        def cp(c, b):
            return pltpu.make_async_copy(
                x_hbm.at[pl.ds(base + c * CH, CH), :],
                bufs[b],
                sems.at[b],
            )

        for k in range(3):
            cp(k, k).start()

        def _chunk(c, b):
            cp(c, b).wait()
            xb = bufs[b]
            for g in range(NG):
                lrows = g * L + lane

                def row_body(r, tvec):
                    row = g * L + r
                    accs = [zero16, zero16, zero16, zero16]
                    for j in range(62):
                        accs[j % 4] = accs[j % 4] + xb[row, pl.ds(j * L, L)]
                    tail = xb[row, pl.ds(984, L)] * tailm
                    acc = (accs[0] + accs[1]) + (accs[2] + accs[3]) + tail
                    t = jnp.sum(acc)
                    oh = (lane == r).astype(jnp.float32)
                    return tvec + t * oh

                tvec = lax.fori_loop(0, L, row_body, zero16)
                off = c * CH + g * L
                a16 = av[pl.ds(off, L)]
                gv = plsc.load_gather(xb, [lrows, a16])
                ov[pl.ds(off, L)] = gv - tvec * jnp.float32(S)

            if c + 3 < NCH:
                cp(c + 3, b).start()

        for c in range(NCH):
            _chunk(c, c % 3)
        pltpu.sync_copy(ov, out_hbm.at[pl.ds(base, PW)])

    return sc_duelqa


_SC = _make_sc()


def kernel(x, a):
    a32 = a.reshape(-1).astype(jnp.int32)
    partial = _SC(x, a32)
    return (partial + x[:, NADV])[:, None]


# speedup vs baseline: 1.0068x; 1.0004x over previous
import functools

import jax
import jax.numpy as jnp
from jax import lax
from jax.experimental import pallas as pl
from jax.experimental.pallas import tpu as pltpu
from jax.experimental.pallas import tpu_sc as plsc

B = 16384
C = 1001
NADV = 1000
S = 1.0 / NADV

NC, NS, L = 2, 16, 16
NW = NC * NS
PW = B // NW
CH = 32
NCH = PW // CH
NG = CH // L


def _make_sc():
    mesh = plsc.VectorSubcoreMesh(core_axis_name="c", subcore_axis_name="s")

    @functools.partial(
        pl.kernel,
        out_type=jax.ShapeDtypeStruct((B,), jnp.float32),
        mesh=mesh,
        compiler_params=pltpu.CompilerParams(needs_layout_passes=False),
        scratch_types=[
            pltpu.VMEM((CH, C), jnp.float32),
            pltpu.VMEM((CH, C), jnp.float32),
            pltpu.VMEM((PW,), jnp.int32),
            pltpu.VMEM((PW,), jnp.float32),
            pltpu.SemaphoreType.DMA((2,)),
            pltpu.SemaphoreType.DMA,
        ],
    )
    def sc_duelqa(x_hbm, a_hbm, out_hbm, xv0, xv1, av, ov, sems, asem):
        wid = lax.axis_index("s") * NC + lax.axis_index("c")
        base = wid * PW
        pltpu.async_copy(a_hbm.at[pl.ds(base, PW)], av, asem).wait()
        lane = lax.iota(jnp.int32, L)
        zero16 = jnp.zeros((L,), jnp.float32)
        tailm = (lane >= 8).astype(jnp.float32)

        def cp(c, b):
            return pltpu.make_async_copy(
                x_hbm.at[pl.ds(base + c * CH, CH), :],
                xv0 if b == 0 else xv1,
                sems.at[b],
            )

        cp(0, 0).start()
        cp(1, 1).start()

        def _chunk(c, b):
            cp(c, b).wait()
            xb = xv0 if b == 0 else xv1
            for g in range(NG):
                lrows = g * L + lane

                def row_body(r, tvec):
                    row = g * L + r
                    accs = [zero16, zero16, zero16, zero16]
                    for j in range(62):
                        accs[j % 4] = accs[j % 4] + xb[row, pl.ds(j * L, L)]
                    tail = xb[row, pl.ds(984, L)] * tailm
                    acc = (accs[0] + accs[1]) + (accs[2] + accs[3]) + tail
                    t = jnp.sum(acc)
                    oh = (lane == r).astype(jnp.float32)
                    return tvec + t * oh

                tvec = lax.fori_loop(0, L, row_body, zero16)
                off = c * CH + g * L
                a16 = av[pl.ds(off, L)]
                gv = plsc.load_gather(xb, [lrows, a16])
                ov[pl.ds(off, L)] = gv - tvec * jnp.float32(S)

            @pl.when(c + 2 < NCH)
            def _():
                cp(c + 2, b).start()

        def pair_body(pair, carry):
            for b in range(2):
                _chunk(pair * 2 + b, b)
            return carry

        lax.fori_loop(0, NCH // 2, pair_body, 0)
        pltpu.sync_copy(ov, out_hbm.at[pl.ds(base, PW)])

    return sc_duelqa


_SC = _make_sc()


def kernel(x, a):
    a32 = a.reshape(-1).astype(jnp.int32)
    partial = _SC(x, a32)
    return (partial + x[:, NADV])[:, None]
